# Initial kernel scaffold; baseline (speedup 1.0000x reference)
import jax
import jax.numpy as jnp
from jax.experimental import pallas as pl


def kernel(h, edge_index, e, snorm_n, W_pre, b_pre, W_post, b_post, W_mix, b_mix):
    raise NotImplementedError("write your pallas kernel here")


# speedup vs baseline: 3.5015x; 3.5015x over previous
import functools

import jax
import jax.numpy as jnp
from jax import lax
from jax.experimental import pallas as pl
from jax.experimental.pallas import tpu as pltpu
from jax.experimental.pallas import tpu_sc as plsc

EPS = 1e-5
AVG_D = 3.5
LEAKY_SLOPE = 0.01

NC, NS, L = 2, 16, 16
NW = NC * NS
RPT = 3
K_RANGES = NW * RPT
R = 105
SUPER = RPT * R
NPAD = K_RANGES * R
CAP = 4096
LISTW = CAP + 64
CE = 3200
CG = 64
D = 128
VR = D // L
SRC_BITS = 14
SRC_MASK = (1 << SRC_BITS) - 1
BIG = 3.0e38


def _sc_segment_stats_body(dst_hbm, src_hbm, a_hbm, c_hbm,
                           summ_hbm, sumsq_hbm, maxm_hbm, minm_hbm, cnt_hbm,
                           dst_st, src_st, plist, elist,
                           acc_s, acc_q, acc_mx, acc_mn, acc_c,
                           sidx, eidx, a_st0, a_st1, c_st0, c_st1,
                           sem_d0, sem_d1, sem_s0, sem_s1,
                           sem_a0, sem_a1, sem_c0, sem_c1):
    cid = lax.axis_index("c")
    sid = lax.axis_index("s")
    wid = cid * NS + sid
    base_node = wid * SUPER
    iota16 = lax.iota(jnp.int32, L)
    zero16i = jnp.zeros((L,), jnp.int32)
    zero16f = jnp.zeros((L,), jnp.float32)
    neg_big = jnp.full((L,), -BIG, jnp.float32)
    pos_big = jnp.full((L,), BIG, jnp.float32)
    one16f = jnp.ones((L,), jnp.float32)

    n_chunks = dst_hbm.shape[0] // CE
    sem_d = (sem_d0, sem_d1)
    sem_s = (sem_s0, sem_s1)
    sem_a = (sem_a0, sem_a1)
    sem_c = (sem_c0, sem_c1)
    a_st = (a_st0, a_st1)
    c_st = (c_st0, c_st1)

    def load_chunk(c, b):
        pltpu.make_async_copy(dst_hbm.at[pl.ds(c * CE, CE)],
                              dst_st.at[pl.ds(b * CE, CE)], sem_d[b]).start()
        pltpu.make_async_copy(src_hbm.at[pl.ds(c * CE, CE)],
                              src_st.at[pl.ds(b * CE, CE)], sem_s[b]).start()

    def wait_chunk(c, b):
        pltpu.make_async_copy(dst_hbm.at[pl.ds(c * CE, CE)],
                              dst_st.at[pl.ds(b * CE, CE)], sem_d[b]).wait()
        pltpu.make_async_copy(src_hbm.at[pl.ds(c * CE, CE)],
                              src_st.at[pl.ds(b * CE, CE)], sem_s[b]).wait()

    load_chunk(0, 0)
    load_chunk(1, 1)

    def process_chunk(c, b, lens):
        def vec_body(v, lens):
            dvec = dst_st[pl.ds(b * CE + v * L, L)]
            svec = src_st[pl.ds(b * CE + v * L, L)]
            dl = dvec - base_node
            packed = (dl << SRC_BITS) | svec
            ev = (c * CE + v * L) + iota16
            out = []
            for r in range(RPT):
                lr = lens[r]
                lo = r * R
                m = (dl >= lo) & (dl < lo + R)
                cs = plsc.cumsum(m.astype(jnp.int32))
                pos = (r * LISTW + lr - 1) + cs
                plsc.store_scatter(plist, [pos], packed, mask=m)
                plsc.store_scatter(elist, [pos], ev, mask=m)
                out.append(jnp.minimum(lr + cs[L - 1], CAP))
            return tuple(out)
        return lax.fori_loop(0, CE // L, vec_body, lens)

    def pair_body(p, lens):
        for b in range(2):
            c = 2 * p + b

            def do(lens=lens, c=c, b=b):
                wait_chunk(c, b)
                new_lens = process_chunk(c, b, lens)

                def prefetch():
                    load_chunk(c + 2, b)
                pl.when(c + 2 < n_chunks)(prefetch)
                return new_lens
            lens = do()
        return lens

    lens = lax.fori_loop(0, n_chunks // 2, pair_body,
                         (jnp.int32(0), jnp.int32(0), jnp.int32(0)))

    for r in range(RPT):
        for k in range(CG // L):
            plist[pl.ds(r * LISTW + lens[r] + k * L, L)] = zero16i
            elist[pl.ds(r * LISTW + lens[r] + k * L, L)] = zero16i

    for r in range(RPT):
        lr = lens[r]

        def zacc(i, _):
            acc_s[pl.ds(i * L, L)] = zero16f
            acc_q[pl.ds(i * L, L)] = zero16f
            acc_mx[pl.ds(i * L, L)] = neg_big
            acc_mn[pl.ds(i * L, L)] = pos_big
            return 0
        lax.fori_loop(0, R * VR, zacc, 0)

        def zcnt(i, _):
            acc_c[pl.ds(i * L, L)] = zero16f
            return 0
        lax.fori_loop(0, R, zcnt, 0)

        n_g = (lr + CG - 1) >> 6

        def extract_idx(ch, b, r=r):
            for k in range(CG // L):
                pv = plist[pl.ds(r * LISTW + ch * CG + k * L, L)]
                sidx[pl.ds(b * CG + k * L, L)] = pv & SRC_MASK
                eidx[pl.ds(b * CG + k * L, L)] = (
                    elist[pl.ds(r * LISTW + ch * CG + k * L, L)])

        def start_gather(b):
            pltpu.make_async_copy(a_hbm.at[sidx.at[pl.ds(b * CG, CG)]],
                                  a_st[b], sem_a[b]).start()
            pltpu.make_async_copy(c_hbm.at[eidx.at[pl.ds(b * CG, CG)]],
                                  c_st[b], sem_c[b]).start()

        def wait_gather(b):
            pltpu.make_async_copy(a_hbm.at[sidx.at[pl.ds(b * CG, CG)]],
                                  a_st[b], sem_a[b]).wait()
            pltpu.make_async_copy(c_hbm.at[eidx.at[pl.ds(b * CG, CG)]],
                                  c_st[b], sem_c[b]).wait()

        def prologue(r=r):
            extract_idx(jnp.int32(0), 0)
            start_gather(0)
        pl.when(n_g > 0)(prologue)

        def g_pair(p, _, r=r, lr=lr, n_g=n_g):
            for b in range(2):
                ch = 2 * p + b

                def do_chunk(ch=ch, b=b, r=r, lr=lr, n_g=n_g):
                    def prefetch(ch=ch, b=b):
                        extract_idx(ch + 1, 1 - b)
                        start_gather(1 - b)
                    pl.when(ch + 1 < n_g)(prefetch)
                    wait_gather(b)
                    n_e = jnp.minimum(lr - ch * CG, CG)

                    def edge_body(i, _, b=b, r=r, ch=ch):
                        pv = plist[pl.ds(r * LISTW + ch * CG + i, L)]
                        pw = pv[0]
                        dl = (pw >> SRC_BITS) - r * R
                        fb = dl * D
                        av_ref = a_st[b]
                        cv_ref = c_st[b]
                        for j in range(VR):
                            av = av_ref[i, pl.ds(j * L, L)]
                            cv = cv_ref[i, pl.ds(j * L, L)]
                            mv = av + cv
                            o = fb + j * L
                            plsc.addupdate(acc_s.at[pl.ds(o, L)], mv)
                            plsc.addupdate(acc_q.at[pl.ds(o, L)], mv * mv)
                            xv = acc_mx[pl.ds(o, L)]
                            acc_mx[pl.ds(o, L)] = jnp.maximum(xv, mv)
                            nv = acc_mn[pl.ds(o, L)]
                            acc_mn[pl.ds(o, L)] = jnp.minimum(nv, mv)
                        plsc.addupdate(acc_c.at[pl.ds(dl * L, L)], one16f)
                        return 0
                    lax.fori_loop(0, n_e, edge_body, 0)
                pl.when(ch < n_g)(do_chunk)
            return 0
        lax.fori_loop(0, (n_g + 1) >> 1, g_pair, 0)

        off = (wid * RPT + r) * R * D
        pltpu.sync_copy(acc_s, summ_hbm.at[pl.ds(off, R * D)])
        pltpu.sync_copy(acc_q, sumsq_hbm.at[pl.ds(off, R * D)])
        pltpu.sync_copy(acc_mx, maxm_hbm.at[pl.ds(off, R * D)])
        pltpu.sync_copy(acc_mn, minm_hbm.at[pl.ds(off, R * D)])
        coff = (wid * RPT + r) * R * L
        pltpu.sync_copy(acc_c, cnt_hbm.at[pl.ds(coff, R * L)])


_sc_segment_stats = pl.kernel(
    _sc_segment_stats_body,
    out_type=[
        jax.ShapeDtypeStruct((NPAD * D,), jnp.float32),
        jax.ShapeDtypeStruct((NPAD * D,), jnp.float32),
        jax.ShapeDtypeStruct((NPAD * D,), jnp.float32),
        jax.ShapeDtypeStruct((NPAD * D,), jnp.float32),
        jax.ShapeDtypeStruct((NPAD * L,), jnp.float32),
    ],
    mesh=plsc.VectorSubcoreMesh(core_axis_name="c", subcore_axis_name="s",
                                num_cores=NC, num_subcores=NS),
    compiler_params=pltpu.CompilerParams(needs_layout_passes=False),
    scratch_types=[
        pltpu.VMEM((2 * CE,), jnp.int32),
        pltpu.VMEM((2 * CE,), jnp.int32),
        pltpu.VMEM((RPT * LISTW,), jnp.int32),
        pltpu.VMEM((RPT * LISTW,), jnp.int32),
        pltpu.VMEM((R * D,), jnp.float32),
        pltpu.VMEM((R * D,), jnp.float32),
        pltpu.VMEM((R * D,), jnp.float32),
        pltpu.VMEM((R * D,), jnp.float32),
        pltpu.VMEM((R * L,), jnp.float32),
        pltpu.VMEM((2 * CG,), jnp.int32),
        pltpu.VMEM((2 * CG,), jnp.int32),
        pltpu.VMEM((CG, D), jnp.float32),
        pltpu.VMEM((CG, D), jnp.float32),
        pltpu.VMEM((CG, D), jnp.float32),
        pltpu.VMEM((CG, D), jnp.float32),
        pltpu.SemaphoreType.DMA,
        pltpu.SemaphoreType.DMA,
        pltpu.SemaphoreType.DMA,
        pltpu.SemaphoreType.DMA,
        pltpu.SemaphoreType.DMA,
        pltpu.SemaphoreType.DMA,
        pltpu.SemaphoreType.DMA,
        pltpu.SemaphoreType.DMA,
    ],
)


def _p0a_body(h_ref, w_ref, a_ref, b_ref):
    ab = jnp.dot(h_ref[...], w_ref[...],
                 preferred_element_type=jnp.float32,
                 precision=lax.Precision.HIGHEST)
    a_ref[...] = ab[:, :D]
    b_ref[...] = ab[:, D:]


def _p0b_body(e_ref, w_ref, bias_ref, c_ref):
    c_ref[...] = jnp.dot(e_ref[...], w_ref[...],
                         preferred_element_type=jnp.float32,
                         precision=lax.Precision.HIGHEST) + bias_ref[...]


def _p3_body(h_ref, b_ref, s_ref, q_ref, mx_ref, mn_ref, c_ref,
             wp_ref, bp_ref, wm_ref, bm_ref, o_ref):
    h = h_ref[...]
    bn = b_ref[...]
    deg = c_ref[..., 0:1]
    degc = jnp.maximum(deg, 1.0)
    has = deg > 0.0
    meanm = s_ref[...] / degc
    mean = meanm + bn
    msqm = q_ref[...] / degc
    var = jnp.maximum(msqm - meanm * meanm, 0.0)
    std = jnp.sqrt(var + EPS)
    mx = mx_ref[...] + bn
    mn = mn_ref[...] + bn
    agg = jnp.concatenate([mean, mx, mn, std], axis=1)
    agg = jnp.where(has, agg, 0.0)
    logd = jnp.log(deg + 1.0)
    amp = logd * (1.0 / AVG_D)
    att = AVG_D / jnp.maximum(logd, EPS)
    hcat = jnp.concatenate([h, agg, agg * amp, agg * att], axis=1)
    h2 = jnp.dot(hcat, wp_ref[...], preferred_element_type=jnp.float32,
                 precision=lax.Precision.HIGHEST) + bp_ref[...]
    hm = jnp.dot(h2, wm_ref[...], preferred_element_type=jnp.float32,
                 precision=lax.Precision.HIGHEST) + bm_ref[...]
    out = jnp.where(hm > 0, hm, LEAKY_SLOPE * hm)
    o_ref[...] = h + out


def kernel(h, edge_index, e, snorm_n, W_pre, b_pre, W_post, b_post,
           W_mix, b_mix):
    n, d = h.shape
    n_edges = e.shape[0]
    src = edge_index[0]
    dst = edge_index[1]
    w12 = jnp.concatenate([W_pre[:d], W_pre[d:2 * d]], axis=1)
    w3 = W_pre[2 * d:]

    bm0 = 2000
    a_mat, b_mat = pl.pallas_call(
        _p0a_body,
        grid=(n // bm0,),
        in_specs=[
            pl.BlockSpec((bm0, d), lambda i: (i, 0)),
            pl.BlockSpec((d, 2 * d), lambda i: (0, 0)),
        ],
        out_specs=[
            pl.BlockSpec((bm0, d), lambda i: (i, 0)),
            pl.BlockSpec((bm0, d), lambda i: (i, 0)),
        ],
        out_shape=[
            jax.ShapeDtypeStruct((n, d), jnp.float32),
            jax.ShapeDtypeStruct((n, d), jnp.float32),
        ],
    )(h, w12)

    bm1 = 8000
    de = e.shape[1]
    c_mat = pl.pallas_call(
        _p0b_body,
        grid=(n_edges // bm1,),
        in_specs=[
            pl.BlockSpec((bm1, de), lambda i: (i, 0)),
            pl.BlockSpec((de, d), lambda i: (0, 0)),
            pl.BlockSpec((1, d), lambda i: (0, 0)),
        ],
        out_specs=pl.BlockSpec((bm1, d), lambda i: (i, 0)),
        out_shape=jax.ShapeDtypeStruct((n_edges, d), jnp.float32),
    )(e, w3, b_pre.reshape(1, d))

    summ, sumsq, maxm, minm, cnt = _sc_segment_stats(dst, src, a_mat, c_mat)

    bm3 = 400
    out = pl.pallas_call(
        _p3_body,
        grid=(n // bm3,),
        in_specs=[
            pl.BlockSpec((bm3, d), lambda i: (i, 0)),
            pl.BlockSpec((bm3, d), lambda i: (i, 0)),
            pl.BlockSpec((bm3, d), lambda i: (i, 0)),
            pl.BlockSpec((bm3, d), lambda i: (i, 0)),
            pl.BlockSpec((bm3, d), lambda i: (i, 0)),
            pl.BlockSpec((bm3, d), lambda i: (i, 0)),
            pl.BlockSpec((bm3, L), lambda i: (i, 0)),
            pl.BlockSpec((13 * d, d), lambda i: (0, 0)),
            pl.BlockSpec((1, d), lambda i: (0, 0)),
            pl.BlockSpec((d, d), lambda i: (0, 0)),
            pl.BlockSpec((1, d), lambda i: (0, 0)),
        ],
        out_specs=pl.BlockSpec((bm3, d), lambda i: (i, 0)),
        out_shape=jax.ShapeDtypeStruct((n, d), jnp.float32),
    )(h, b_mat, summ.reshape(NPAD, D), sumsq.reshape(NPAD, D),
      maxm.reshape(NPAD, D), minm.reshape(NPAD, D), cnt.reshape(NPAD, L),
      W_post, b_post.reshape(1, d), W_mix, b_mix.reshape(1, d))
    return out


# speedup vs baseline: 3.6431x; 1.0404x over previous
import functools

import jax
import jax.numpy as jnp
from jax import lax
from jax.experimental import pallas as pl
from jax.experimental.pallas import tpu as pltpu
from jax.experimental.pallas import tpu_sc as plsc

EPS = 1e-5
AVG_D = 3.5
LEAKY_SLOPE = 0.01

NC, NS, L = 2, 16, 16
NW = NC * NS
RPT = 3
K_RANGES = NW * RPT
R = 105
SUPER = RPT * R
NPAD = K_RANGES * R
CAP = 4096
LISTW = CAP + 64
CE = 3200
CG = 64
D = 128
VR = D // L
SRC_BITS = 14
SRC_MASK = (1 << SRC_BITS) - 1
R2 = R + 1
BIG = 3.0e38


def _sc_segment_stats_body(dst_hbm, src_hbm, a_hbm, c_hbm,
                           summ_hbm, sumsq_hbm, maxm_hbm, minm_hbm, cnt_hbm,
                           dst_st, src_st, plist, elist,
                           acc_s, acc_q, acc_mx, acc_mn, acc_c,
                           sidx, eidx, a_st0, a_st1, c_st0, c_st1, lens_sm,
                           sem_d0, sem_d1, sem_s0, sem_s1,
                           sem_a0, sem_a1, sem_c0, sem_c1):
    cid = lax.axis_index("c")
    sid = lax.axis_index("s")
    wid = cid * NS + sid
    base_node = wid * SUPER
    iota16 = lax.iota(jnp.int32, L)
    zero16i = jnp.zeros((L,), jnp.int32)
    zero16f = jnp.zeros((L,), jnp.float32)
    neg_big = jnp.full((L,), -BIG, jnp.float32)
    pos_big = jnp.full((L,), BIG, jnp.float32)
    one16f = jnp.ones((L,), jnp.float32)

    n_chunks = dst_hbm.shape[0] // CE
    sem_d = (sem_d0, sem_d1)
    sem_s = (sem_s0, sem_s1)
    sem_a = (sem_a0, sem_a1)
    sem_c = (sem_c0, sem_c1)
    a_st = (a_st0, a_st1)
    c_st = (c_st0, c_st1)

    def load_chunk(c, b):
        pltpu.make_async_copy(dst_hbm.at[pl.ds(c * CE, CE)],
                              dst_st.at[pl.ds(b * CE, CE)], sem_d[b]).start()
        pltpu.make_async_copy(src_hbm.at[pl.ds(c * CE, CE)],
                              src_st.at[pl.ds(b * CE, CE)], sem_s[b]).start()

    def wait_chunk(c, b):
        pltpu.make_async_copy(dst_hbm.at[pl.ds(c * CE, CE)],
                              dst_st.at[pl.ds(b * CE, CE)], sem_d[b]).wait()
        pltpu.make_async_copy(src_hbm.at[pl.ds(c * CE, CE)],
                              src_st.at[pl.ds(b * CE, CE)], sem_s[b]).wait()

    load_chunk(0, 0)
    load_chunk(1, 1)

    def process_chunk(c, b, lens):
        def vec4_body(v4, lens):
            for u in range(4):
                v = v4 * 4 + u
                dvec = dst_st[pl.ds(b * CE + v * L, L)]
                svec = src_st[pl.ds(b * CE + v * L, L)]
                dl = dvec - base_node
                packed = (dl << (SRC_BITS + 7)) | svec
                ev = (c * CE + v * L) + iota16
                out = []
                for r in range(RPT):
                    lr = lens[r]
                    lo = r * R
                    m = (dl >= lo) & (dl < lo + R)
                    cs = plsc.cumsum(m.astype(jnp.int32))
                    pos = (r * LISTW + lr - 1) + cs
                    plsc.store_scatter(plist, [pos], packed, mask=m)
                    plsc.store_scatter(elist, [pos], ev, mask=m)
                    out.append(jnp.minimum(lr + cs[L - 1], CAP))
                lens = tuple(out)
            return lens
        return lax.fori_loop(0, CE // L // 4, vec4_body, lens)

    def pair_body(p, lens):
        for b in range(2):
            c = 2 * p + b

            def do(lens=lens, c=c, b=b):
                wait_chunk(c, b)
                new_lens = process_chunk(c, b, lens)

                def prefetch():
                    load_chunk(c + 2, b)
                pl.when(c + 2 < n_chunks)(prefetch)
                return new_lens
            lens = do()
        return lens

    lens = lax.fori_loop(0, n_chunks // 2, pair_body,
                         (jnp.int32(0), jnp.int32(0), jnp.int32(0)))

    for r in range(RPT):
        pad = jnp.full((L,), ((r * R + R) * D) << SRC_BITS, jnp.int32)
        for k in range(CG // L):
            plist[pl.ds(r * LISTW + lens[r] + k * L, L)] = pad
            elist[pl.ds(r * LISTW + lens[r] + k * L, L)] = zero16i
        lens_sm[r] = lens[r]

    def range_body(r, _):
        lr = lens_sm[r]

        def zacc(i, _):
            acc_s[pl.ds(i * L, L)] = zero16f
            acc_q[pl.ds(i * L, L)] = zero16f
            acc_mx[pl.ds(i * L, L)] = neg_big
            acc_mn[pl.ds(i * L, L)] = pos_big
            return 0
        lax.fori_loop(0, R2 * VR, zacc, 0)

        def zcnt(i, _):
            acc_c[pl.ds(i * L, L)] = zero16f
            return 0
        lax.fori_loop(0, R2, zcnt, 0)

        n_g = (lr + CG - 1) >> 6

        def extract_idx(ch, b):
            for k in range(CG // L):
                pv = plist[pl.ds(r * LISTW + ch * CG + k * L, L)]
                sidx[pl.ds(b * CG + k * L, L)] = pv & SRC_MASK
                eidx[pl.ds(b * CG + k * L, L)] = (
                    elist[pl.ds(r * LISTW + ch * CG + k * L, L)])

        def start_gather(b):
            pltpu.make_async_copy(a_hbm.at[sidx.at[pl.ds(b * CG, CG)]],
                                  a_st[b], sem_a[b]).start()
            pltpu.make_async_copy(c_hbm.at[eidx.at[pl.ds(b * CG, CG)]],
                                  c_st[b], sem_c[b]).start()

        def wait_gather(b):
            pltpu.make_async_copy(a_hbm.at[sidx.at[pl.ds(b * CG, CG)]],
                                  a_st[b], sem_a[b]).wait()
            pltpu.make_async_copy(c_hbm.at[eidx.at[pl.ds(b * CG, CG)]],
                                  c_st[b], sem_c[b]).wait()

        def prologue():
            extract_idx(jnp.int32(0), 0)
            start_gather(0)
        pl.when(n_g > 0)(prologue)

        def g_pair(p, _):
            for b in range(2):
                ch = 2 * p + b

                def do_chunk(ch=ch, b=b):
                    def prefetch(ch=ch, b=b):
                        extract_idx(ch + 1, 1 - b)
                        start_gather(1 - b)
                    pl.when(ch + 1 < n_g)(prefetch)
                    wait_gather(b)
                    av_ref = a_st[b]
                    cv_ref = c_st[b]

                    def grp_body(g, _, ch=ch,
                                 av_ref=av_ref, cv_ref=cv_ref):
                        pv = plist[pl.ds(r * LISTW + ch * CG + g * L, L)]
                        row0 = g * L
                        for t in range(L):
                            pw = pv[t]
                            fb = (pw >> SRC_BITS) - r * R * D
                            for j in range(VR):
                                av = av_ref[row0 + t, pl.ds(j * L, L)]
                                cv = cv_ref[row0 + t, pl.ds(j * L, L)]
                                mv = av + cv
                                o = fb + j * L
                                plsc.addupdate(acc_s.at[pl.ds(o, L)], mv)
                                plsc.addupdate(acc_q.at[pl.ds(o, L)], mv * mv)
                                xv = acc_mx[pl.ds(o, L)]
                                acc_mx[pl.ds(o, L)] = jnp.maximum(xv, mv)
                                nv = acc_mn[pl.ds(o, L)]
                                acc_mn[pl.ds(o, L)] = jnp.minimum(nv, mv)
                            plsc.addupdate(acc_c.at[pl.ds(fb >> 3, L)],
                                           one16f)
                        return 0
                    lax.fori_loop(0, CG // L, grp_body, 0)
                pl.when(ch < n_g)(do_chunk)
            return 0
        lax.fori_loop(0, (n_g + 1) >> 1, g_pair, 0)

        off = (wid * RPT + r) * R * D
        pltpu.sync_copy(acc_s.at[pl.ds(0, R * D)],
                        summ_hbm.at[pl.ds(off, R * D)])
        pltpu.sync_copy(acc_q.at[pl.ds(0, R * D)],
                        sumsq_hbm.at[pl.ds(off, R * D)])
        pltpu.sync_copy(acc_mx.at[pl.ds(0, R * D)],
                        maxm_hbm.at[pl.ds(off, R * D)])
        pltpu.sync_copy(acc_mn.at[pl.ds(0, R * D)],
                        minm_hbm.at[pl.ds(off, R * D)])
        coff = (wid * RPT + r) * R * L
        pltpu.sync_copy(acc_c.at[pl.ds(0, R * L)],
                        cnt_hbm.at[pl.ds(coff, R * L)])
        return 0

    lax.fori_loop(0, RPT, range_body, 0)


_sc_segment_stats = pl.kernel(
    _sc_segment_stats_body,
    out_type=[
        jax.ShapeDtypeStruct((NPAD * D,), jnp.float32),
        jax.ShapeDtypeStruct((NPAD * D,), jnp.float32),
        jax.ShapeDtypeStruct((NPAD * D,), jnp.float32),
        jax.ShapeDtypeStruct((NPAD * D,), jnp.float32),
        jax.ShapeDtypeStruct((NPAD * L,), jnp.float32),
    ],
    mesh=plsc.VectorSubcoreMesh(core_axis_name="c", subcore_axis_name="s",
                                num_cores=NC, num_subcores=NS),
    compiler_params=pltpu.CompilerParams(needs_layout_passes=False),
    scratch_types=[
        pltpu.VMEM((2 * CE,), jnp.int32),
        pltpu.VMEM((2 * CE,), jnp.int32),
        pltpu.VMEM((RPT * LISTW,), jnp.int32),
        pltpu.VMEM((RPT * LISTW,), jnp.int32),
        pltpu.VMEM((R2 * D,), jnp.float32),
        pltpu.VMEM((R2 * D,), jnp.float32),
        pltpu.VMEM((R2 * D,), jnp.float32),
        pltpu.VMEM((R2 * D,), jnp.float32),
        pltpu.VMEM((R2 * L,), jnp.float32),
        pltpu.VMEM((2 * CG,), jnp.int32),
        pltpu.VMEM((2 * CG,), jnp.int32),
        pltpu.VMEM((CG, D), jnp.float32),
        pltpu.VMEM((CG, D), jnp.float32),
        pltpu.VMEM((CG, D), jnp.float32),
        pltpu.VMEM((CG, D), jnp.float32),
        pltpu.SMEM((8,), jnp.int32),
        pltpu.SemaphoreType.DMA,
        pltpu.SemaphoreType.DMA,
        pltpu.SemaphoreType.DMA,
        pltpu.SemaphoreType.DMA,
        pltpu.SemaphoreType.DMA,
        pltpu.SemaphoreType.DMA,
        pltpu.SemaphoreType.DMA,
        pltpu.SemaphoreType.DMA,
    ],
)


def _p0a_body(h_ref, w_ref, a_ref, b_ref):
    ab = jnp.dot(h_ref[...], w_ref[...],
                 preferred_element_type=jnp.float32,
                 precision=lax.Precision.HIGHEST)
    a_ref[...] = ab[:, :D]
    b_ref[...] = ab[:, D:]


def _p0b_body(e_ref, w_ref, bias_ref, c_ref):
    c_ref[...] = jnp.dot(e_ref[...], w_ref[...],
                         preferred_element_type=jnp.float32,
                         precision=lax.Precision.HIGHEST) + bias_ref[...]


def _p3_body(h_ref, b_ref, s_ref, q_ref, mx_ref, mn_ref, c_ref,
             wp_ref, bp_ref, wm_ref, bm_ref, o_ref):
    h = h_ref[...]
    bn = b_ref[...]
    deg = c_ref[..., 0:1]
    degc = jnp.maximum(deg, 1.0)
    has = deg > 0.0
    meanm = s_ref[...] / degc
    mean = meanm + bn
    msqm = q_ref[...] / degc
    var = jnp.maximum(msqm - meanm * meanm, 0.0)
    std = jnp.sqrt(var + EPS)
    mx = mx_ref[...] + bn
    mn = mn_ref[...] + bn
    agg = jnp.concatenate([mean, mx, mn, std], axis=1)
    agg = jnp.where(has, agg, 0.0)
    logd = jnp.log(deg + 1.0)
    amp = logd * (1.0 / AVG_D)
    att = AVG_D / jnp.maximum(logd, EPS)
    hcat = jnp.concatenate([h, agg, agg * amp, agg * att], axis=1)
    h2 = jnp.dot(hcat, wp_ref[...], preferred_element_type=jnp.float32,
                 precision=lax.Precision.HIGHEST) + bp_ref[...]
    hm = jnp.dot(h2, wm_ref[...], preferred_element_type=jnp.float32,
                 precision=lax.Precision.HIGHEST) + bm_ref[...]
    out = jnp.where(hm > 0, hm, LEAKY_SLOPE * hm)
    o_ref[...] = h + out


def kernel(h, edge_index, e, snorm_n, W_pre, b_pre, W_post, b_post,
           W_mix, b_mix):
    n, d = h.shape
    n_edges = e.shape[0]
    src = edge_index[0]
    dst = edge_index[1]
    w12 = jnp.concatenate([W_pre[:d], W_pre[d:2 * d]], axis=1)
    w3 = W_pre[2 * d:]

    bm0 = 2000
    a_mat, b_mat = pl.pallas_call(
        _p0a_body,
        grid=(n // bm0,),
        in_specs=[
            pl.BlockSpec((bm0, d), lambda i: (i, 0)),
            pl.BlockSpec((d, 2 * d), lambda i: (0, 0)),
        ],
        out_specs=[
            pl.BlockSpec((bm0, d), lambda i: (i, 0)),
            pl.BlockSpec((bm0, d), lambda i: (i, 0)),
        ],
        out_shape=[
            jax.ShapeDtypeStruct((n, d), jnp.float32),
            jax.ShapeDtypeStruct((n, d), jnp.float32),
        ],
    )(h, w12)

    bm1 = 8000
    de = e.shape[1]
    c_mat = pl.pallas_call(
        _p0b_body,
        grid=(n_edges // bm1,),
        in_specs=[
            pl.BlockSpec((bm1, de), lambda i: (i, 0)),
            pl.BlockSpec((de, d), lambda i: (0, 0)),
            pl.BlockSpec((1, d), lambda i: (0, 0)),
        ],
        out_specs=pl.BlockSpec((bm1, d), lambda i: (i, 0)),
        out_shape=jax.ShapeDtypeStruct((n_edges, d), jnp.float32),
    )(e, w3, b_pre.reshape(1, d))

    summ, sumsq, maxm, minm, cnt = _sc_segment_stats(dst, src, a_mat, c_mat)

    bm3 = 400
    out = pl.pallas_call(
        _p3_body,
        grid=(n // bm3,),
        in_specs=[
            pl.BlockSpec((bm3, d), lambda i: (i, 0)),
            pl.BlockSpec((bm3, d), lambda i: (i, 0)),
            pl.BlockSpec((bm3, d), lambda i: (i, 0)),
            pl.BlockSpec((bm3, d), lambda i: (i, 0)),
            pl.BlockSpec((bm3, d), lambda i: (i, 0)),
            pl.BlockSpec((bm3, d), lambda i: (i, 0)),
            pl.BlockSpec((bm3, L), lambda i: (i, 0)),
            pl.BlockSpec((13 * d, d), lambda i: (0, 0)),
            pl.BlockSpec((1, d), lambda i: (0, 0)),
            pl.BlockSpec((d, d), lambda i: (0, 0)),
            pl.BlockSpec((1, d), lambda i: (0, 0)),
        ],
        out_specs=pl.BlockSpec((bm3, d), lambda i: (i, 0)),
        out_shape=jax.ShapeDtypeStruct((n, d), jnp.float32),
    )(h, b_mat, summ.reshape(NPAD, D), sumsq.reshape(NPAD, D),
      maxm.reshape(NPAD, D), minm.reshape(NPAD, D), cnt.reshape(NPAD, L),
      W_post, b_post.reshape(1, d), W_mix, b_mix.reshape(1, d))
    return out
